# Initial kernel scaffold; baseline (speedup 1.0000x reference)
import jax
import jax.numpy as jnp
from jax.experimental import pallas as pl


def kernel(x, edge_index, W1, b1, W2, b2, W3, b3, Wout, bout):
    raise NotImplementedError("write your pallas kernel here")


# speedup vs baseline: 2.6639x; 2.6639x over previous
import functools

import jax
import jax.numpy as jnp
from jax.experimental import pallas as pl


_MT = 256
_KT = 1024


def _agg_mm_kernel(a_ref, b_ref, bias_ref, o_ref, *, nk, add_bias):
    k = pl.program_id(1)

    @pl.when(k == 0)
    def _():
        o_ref[...] = jnp.zeros_like(o_ref)

    o_ref[...] += jnp.dot(a_ref[...], b_ref[...],
                          preferred_element_type=jnp.float32)

    if add_bias:
        @pl.when(k == nk - 1)
        def _():
            o_ref[...] += bias_ref[...]


def _dense_mm_kernel(a_ref, w_ref, bias_ref, o_ref, *, activate):
    acc = jnp.dot(a_ref[...], w_ref[...], preferred_element_type=jnp.float32)
    if activate:
        h = jnp.maximum(acc + bias_ref[...], 0.0)
        f = h.shape[1]
        mean = jnp.mean(h, axis=1, keepdims=True)
        var = jnp.sum((h - mean) ** 2, axis=1, keepdims=True) / (f - 1)
        o_ref[...] = (h - mean) / (jnp.sqrt(var) + 1e-5)
    else:
        o_ref[...] = acc


def _agg_mm(a, b, bias, add_bias):
    np_, _ = a.shape
    f = b.shape[1]
    if bias is None:
        bias = jnp.zeros((1, f), jnp.float32)
    nm, nk = np_ // _MT, np_ // _KT
    return pl.pallas_call(
        functools.partial(_agg_mm_kernel, nk=nk, add_bias=add_bias),
        grid=(nm, nk),
        in_specs=[
            pl.BlockSpec((_MT, _KT), lambda m, k: (m, k)),
            pl.BlockSpec((_KT, f), lambda m, k: (k, 0)),
            pl.BlockSpec((1, f), lambda m, k: (0, 0)),
        ],
        out_specs=pl.BlockSpec((_MT, f), lambda m, k: (m, 0)),
        out_shape=jax.ShapeDtypeStruct((np_, f), jnp.float32),
    )(a, b, bias)


def _dense_mm(a, w, bias, activate):
    np_, kdim = a.shape
    f = w.shape[1]
    nm = np_ // _MT
    return pl.pallas_call(
        functools.partial(_dense_mm_kernel, activate=activate),
        grid=(nm,),
        in_specs=[
            pl.BlockSpec((_MT, kdim), lambda m: (m, 0)),
            pl.BlockSpec((kdim, f), lambda m: (0, 0)),
            pl.BlockSpec((1, f), lambda m: (0, 0)),
        ],
        out_specs=pl.BlockSpec((_MT, f), lambda m: (m, 0)),
        out_shape=jax.ShapeDtypeStruct((np_, f), jnp.float32),
    )(a, w, bias)


def kernel(x, edge_index, W1, b1, W2, b2, W3, b3, Wout, bout):
    n, din = x.shape
    src = edge_index[0].astype(jnp.int32)
    dst = edge_index[1].astype(jnp.int32)

    deg = jnp.ones((n,), jnp.float32).at[dst].add(
        jnp.ones(src.shape, jnp.float32))
    dis = jax.lax.rsqrt(deg)
    norm = dis[src] * dis[dst]

    np_ = ((n + _KT - 1) // _KT) * _KT
    a_hat = jnp.zeros((np_, np_), jnp.float32)
    a_hat = a_hat.at[dst, src].add(norm)
    diag = jnp.arange(n, dtype=jnp.int32)
    a_hat = a_hat.at[diag, diag].add(1.0 / deg)

    xp = jnp.zeros((np_, din), x.dtype).at[:n].set(x)

    ax = _agg_mm(a_hat, xp, None, add_bias=False)
    h = _dense_mm(ax, W1, b1[None, :], activate=True)
    for (w, b) in ((W2, b2), (W3, b3)):
        ah = _agg_mm(a_hat, h, None, add_bias=False)
        h = _dense_mm(ah, w, b[None, :], activate=True)
    penultimate = h[:n]
    g = _dense_mm(h, Wout, jnp.zeros((1, Wout.shape[1]), jnp.float32),
                  activate=False)
    out = _agg_mm(a_hat, g, bout[None, :], add_bias=True)
    return (out[:n], penultimate)
